# Initial kernel scaffold; baseline (speedup 1.0000x reference)
import jax
import jax.numpy as jnp
from jax.experimental import pallas as pl


def kernel(x, Wg1, Wg2, W_gate, W_up, W_down):
    raise NotImplementedError("write your pallas kernel here")


# speedup vs baseline: 1.3684x; 1.3684x over previous
import functools

import jax
import jax.numpy as jnp
from jax.experimental import pallas as pl

INPUT_SIZE = 1024
HIDDEN_SIZE = 4096
OUTPUT_SIZE = 1024
NUM_EXPERTS = 8
NUM_SELECTS = 2
H_PER_EXPERT = HIDDEN_SIZE // NUM_EXPERTS

_GATE_BT = 512
_FFN_BT = 512


def _gate_kernel(x_ref, wg1_ref, wg2_ref, w_ref, imp_ref, cnt_ref, loss_ref):
    i = pl.program_id(0)
    nb = pl.num_programs(0)
    xb = x_ref[...]
    h = jnp.tanh(jax.lax.dot_general(xb, wg1_ref[...],
                                     (((1,), (1,)), ((), ())),
                                     preferred_element_type=jnp.float32))
    logits = jax.lax.dot_general(h, wg2_ref[...],
                                 (((1,), (1,)), ((), ())),
                                 preferred_element_type=jnp.float32)
    lane = jax.lax.broadcasted_iota(jnp.int32, logits.shape, 1)
    m1 = jnp.max(logits, axis=1, keepdims=True)
    i1 = jnp.min(jnp.where(logits == m1, lane, NUM_EXPERTS), axis=1, keepdims=True)
    masked = jnp.where(lane == i1, -jnp.inf, logits)
    m2 = jnp.max(masked, axis=1, keepdims=True)
    i2 = jnp.min(jnp.where(masked == m2, lane, NUM_EXPERTS), axis=1, keepdims=True)
    e2 = jnp.exp(m2 - m1)
    denom = 1.0 + e2
    s1 = 1.0 / denom
    s2 = e2 / denom
    sel1 = lane == i1
    sel2 = lane == i2
    w = jnp.where(sel1, s1, 0.0) + jnp.where(sel2, s2, 0.0)
    w_ref[...] = w
    imp_part = jnp.sum(w, axis=0, keepdims=True)
    cnt_part = jnp.sum(sel1.astype(jnp.int32) + sel2.astype(jnp.int32),
                       axis=0, keepdims=True)

    @pl.when(i == 0)
    def _():
        imp_ref[...] = jnp.zeros_like(imp_ref)
        cnt_ref[...] = jnp.zeros_like(cnt_ref)

    imp_ref[0:1, :] += imp_part
    cnt_ref[0:1, :] += cnt_part

    @pl.when(i == nb - 1)
    def _():
        imp = imp_ref[0:1, :]
        cnt = cnt_ref[0:1, :].astype(jnp.float32)

        def cv2(v):
            mean = jnp.sum(v) / NUM_EXPERTS
            var = jnp.sum((v - mean) ** 2) / (NUM_EXPERTS - 1)
            return var / (mean * mean + 1e-10)

        loss_ref[...] = jnp.full_like(loss_ref, 0.01 * (cv2(imp) + cv2(cnt)))


def _ffn_kernel(x_ref, w_ref, wg_ref, wu_ref, wd_ref, y_ref):
    e = pl.program_id(1)
    xb = x_ref[...]
    hg = jax.lax.dot_general(xb, wg_ref[0], (((1,), (1,)), ((), ())),
                             preferred_element_type=jnp.float32)
    hu = jax.lax.dot_general(xb, wu_ref[0], (((1,), (1,)), ((), ())),
                             preferred_element_type=jnp.float32)
    h = (hg * jax.nn.sigmoid(hg) * hu).astype(jnp.bfloat16)
    o = jax.lax.dot_general(h, wd_ref[0], (((1,), (1,)), ((), ())),
                            preferred_element_type=jnp.float32)
    lane = jax.lax.broadcasted_iota(jnp.int32, w_ref.shape, 1)
    wcol = jnp.sum(jnp.where(lane == e, w_ref[...], 0.0), axis=1, keepdims=True)
    contrib = wcol * o

    @pl.when(e == 0)
    def _():
        y_ref[...] = contrib

    @pl.when(e > 0)
    def _():
        y_ref[...] += contrib


@jax.jit
def kernel(x, Wg1, Wg2, W_gate, W_up, W_down):
    B, S, D = x.shape
    xf = x.reshape(-1, D)
    T = xf.shape[0]
    E = NUM_EXPERTS

    nb_gate = T // _GATE_BT
    w, imp, cnt, loss = pl.pallas_call(
        _gate_kernel,
        grid=(nb_gate,),
        in_specs=[
            pl.BlockSpec((_GATE_BT, D), lambda i: (i, 0)),
            pl.BlockSpec((E, D), lambda i: (0, 0)),
            pl.BlockSpec((E, E), lambda i: (0, 0)),
        ],
        out_specs=[
            pl.BlockSpec((_GATE_BT, E), lambda i: (i, 0)),
            pl.BlockSpec((8, E), lambda i: (0, 0)),
            pl.BlockSpec((8, E), lambda i: (0, 0)),
            pl.BlockSpec((8, E), lambda i: (0, 0)),
        ],
        out_shape=[
            jax.ShapeDtypeStruct((T, E), jnp.float32),
            jax.ShapeDtypeStruct((8, E), jnp.float32),
            jax.ShapeDtypeStruct((8, E), jnp.int32),
            jax.ShapeDtypeStruct((8, E), jnp.float32),
        ],
    )(xf, Wg1, Wg2)

    importance = imp[0]
    load = cnt[0]
    balance_loss = loss[0, 0]

    xb16 = xf.astype(jnp.bfloat16)
    wg16 = W_gate.astype(jnp.bfloat16)
    wu16 = W_up.astype(jnp.bfloat16)
    wd16 = W_down.astype(jnp.bfloat16)

    nb_ffn = T // _FFN_BT
    y = pl.pallas_call(
        _ffn_kernel,
        grid=(nb_ffn, E),
        in_specs=[
            pl.BlockSpec((_FFN_BT, D), lambda i, e: (i, 0)),
            pl.BlockSpec((_FFN_BT, E), lambda i, e: (i, 0)),
            pl.BlockSpec((1, H_PER_EXPERT, D), lambda i, e: (e, 0, 0)),
            pl.BlockSpec((1, H_PER_EXPERT, D), lambda i, e: (e, 0, 0)),
            pl.BlockSpec((1, OUTPUT_SIZE, H_PER_EXPERT), lambda i, e: (e, 0, 0)),
        ],
        out_specs=pl.BlockSpec((_FFN_BT, OUTPUT_SIZE), lambda i, e: (i, 0)),
        out_shape=jax.ShapeDtypeStruct((T, OUTPUT_SIZE), jnp.float32),
    )(xb16, w, wg16, wu16, wd16)

    return (y.reshape(B, S, OUTPUT_SIZE), balance_loss, load, importance)
